# Initial kernel scaffold; baseline (speedup 1.0000x reference)
import jax
import jax.numpy as jnp
from jax.experimental import pallas as pl


def kernel(x, edge_index, W1, b1, W2, b2):
    raise NotImplementedError("write your pallas kernel here")


# speedup vs baseline: 3.0655x; 3.0655x over previous
import functools

import jax
import jax.numpy as jnp
from jax import lax
from jax.experimental import pallas as pl
from jax.experimental.pallas import tpu as pltpu
from jax.experimental.pallas import tpu_sc as plsc

IN_DIM = 128
HID = 128
N_NODES = 10000
N_EDGES = 320000

NC = 2
NS = 16
L = 16
NW = NC * NS
PER_W = N_EDGES // NW
K = 128
N_CHUNKS = -(-PER_W // K)


def _tables_body(x_ref, w1a_ref, w1b_ref, b1_ref, a_ref, b_ref):
    x = x_ref[...]
    dn = (((1,), (1,)), ((), ()))
    a = lax.dot_general(x, w1a_ref[...], dn, preferred_element_type=jnp.float32)
    b = lax.dot_general(x, w1b_ref[...], dn, preferred_element_type=jnp.float32)
    a_ref[...] = a + b1_ref[...]
    b_ref[...] = b


def _make_tables(x, w1a, w1b, b1):
    blk = 1000
    grid = (N_NODES // blk,)
    return pl.pallas_call(
        _tables_body,
        grid=grid,
        in_specs=[
            pl.BlockSpec((blk, IN_DIM), lambda i: (i, 0)),
            pl.BlockSpec((HID, IN_DIM), lambda i: (0, 0)),
            pl.BlockSpec((HID, IN_DIM), lambda i: (0, 0)),
            pl.BlockSpec((1, HID), lambda i: (0, 0)),
        ],
        out_specs=[
            pl.BlockSpec((blk, HID), lambda i: (i, 0)),
            pl.BlockSpec((blk, HID), lambda i: (i, 0)),
        ],
        out_shape=[
            jax.ShapeDtypeStruct((N_NODES, HID), jnp.float32),
            jax.ShapeDtypeStruct((N_NODES, HID), jnp.float32),
        ],
    )(x, w1a, w1b, b1)


def _edge_body(a_hbm, b_hbm, row_hbm, col_hbm, w2_hbm, b2_hbm, out_hbm,
               ridx, cidx, arows, brows, hsum, outv, w2v, b2v, sem_a, sem_b):
    wid = lax.axis_index("s") * NC + lax.axis_index("c")
    base = pl.multiple_of(wid * PER_W, 8)

    pltpu.sync_copy(w2_hbm, w2v)
    pltpu.sync_copy(b2_hbm, b2v)
    iota = lax.iota(jnp.int32, L)

    def chunk(c, _):
        start = pl.multiple_of(
            base + jnp.minimum(c * K, PER_W - K), 8)
        pltpu.sync_copy(row_hbm.at[pl.ds(start, K)], ridx)
        pltpu.sync_copy(col_hbm.at[pl.ds(start, K)], cidx)
        pltpu.async_copy(a_hbm.at[ridx], arows, sem_a).wait()
        pltpu.async_copy(b_hbm.at[cidx], brows, sem_b).wait()

        def edge(e, _):
            acc = b2v[...]
            for ci in range(HID // L):
                a = arows[e, pl.ds(ci * L, L)]
                b = brows[e, pl.ds(ci * L, L)]
                h = jnp.maximum(a + b, 0.0)
                acc = acc + h * w2v[ci]
            hsum[pl.ds(e * L, L)] = acc
            return 0

        lax.fori_loop(0, K, edge, 0)

        for g in range(K // L):
            flat = (g * L + iota) * L
            z = plsc.load_gather(hsum, [flat])
            for j in range(1, L):
                z = z + plsc.load_gather(hsum, [flat + j])
            outv[pl.ds(g * L, L)] = 1.0 / (1.0 + jnp.exp(-z))

        pltpu.sync_copy(outv, out_hbm.at[pl.ds(start, K)])
        return 0

    lax.fori_loop(0, N_CHUNKS, chunk, 0)


_edge_kernel = functools.partial(
    pl.kernel,
    out_type=jax.ShapeDtypeStruct((N_EDGES,), jnp.float32),
    mesh=plsc.VectorSubcoreMesh(core_axis_name="c", subcore_axis_name="s"),
    scratch_types=[
        pltpu.VMEM((K,), jnp.int32),
        pltpu.VMEM((K,), jnp.int32),
        pltpu.VMEM((K, HID), jnp.float32),
        pltpu.VMEM((K, HID), jnp.float32),
        pltpu.VMEM((K * L,), jnp.float32),
        pltpu.VMEM((K,), jnp.float32),
        pltpu.VMEM((HID // L, L), jnp.float32),
        pltpu.VMEM((L,), jnp.float32),
        pltpu.SemaphoreType.DMA,
        pltpu.SemaphoreType.DMA,
    ],
    compiler_params=pltpu.CompilerParams(needs_layout_passes=False),
)(_edge_body)


def kernel(x, edge_index, W1, b1, W2, b2):
    row = edge_index[0].astype(jnp.int32)
    col = edge_index[1].astype(jnp.int32)
    w1a = W1[:, :IN_DIM]
    w1b = W1[:, IN_DIM:]
    a_tab, b_tab = _make_tables(x, w1a, w1b, b1.reshape(1, HID))
    w2r = W2.reshape(HID // L, L).astype(jnp.float32)
    b2v = jnp.broadcast_to(b2 / jnp.float32(L), (L,)).astype(jnp.float32)
    return _edge_kernel(a_tab, b_tab, row, col, w2r, b2v)


# speedup vs baseline: 4.6099x; 1.5038x over previous
import functools

import jax
import jax.numpy as jnp
from jax import lax
from jax.experimental import pallas as pl
from jax.experimental.pallas import tpu as pltpu
from jax.experimental.pallas import tpu_sc as plsc

IN_DIM = 128
HID = 128
N_NODES = 10000
N_EDGES = 320000

NC = 2
NS = 16
L = 16
NW = NC * NS
PER_W = N_EDGES // NW
K = 128
N_CHUNKS = -(-PER_W // K)


def _tables_body(x_ref, w1a_ref, w1b_ref, b1_ref, a_ref, b_ref):
    x = x_ref[...]
    dn = (((1,), (1,)), ((), ()))
    a = lax.dot_general(x, w1a_ref[...], dn, preferred_element_type=jnp.float32)
    b = lax.dot_general(x, w1b_ref[...], dn, preferred_element_type=jnp.float32)
    a_ref[...] = a + b1_ref[...]
    b_ref[...] = b


def _make_tables(x, w1a, w1b, b1):
    blk = 1000
    grid = (N_NODES // blk,)
    return pl.pallas_call(
        _tables_body,
        grid=grid,
        in_specs=[
            pl.BlockSpec((blk, IN_DIM), lambda i: (i, 0)),
            pl.BlockSpec((HID, IN_DIM), lambda i: (0, 0)),
            pl.BlockSpec((HID, IN_DIM), lambda i: (0, 0)),
            pl.BlockSpec((1, HID), lambda i: (0, 0)),
        ],
        out_specs=[
            pl.BlockSpec((blk, HID), lambda i: (i, 0)),
            pl.BlockSpec((blk, HID), lambda i: (i, 0)),
        ],
        out_shape=[
            jax.ShapeDtypeStruct((N_NODES, HID), jnp.float32),
            jax.ShapeDtypeStruct((N_NODES, HID), jnp.float32),
        ],
    )(x, w1a, w1b, b1)


N_CH = 2 * (-(-PER_W // (2 * K)))


def _edge_body(a_hbm, b_hbm, row_hbm, col_hbm, w2_hbm, b2_hbm, out_hbm,
               ridx0, ridx1, cidx0, cidx1, ar0, ar1, br0, br1,
               ov0, ov1, hsum, w2v, b2v,
               sa0, sa1, sb0, sb1, so0, so1):
    ridx = (ridx0, ridx1)
    cidx = (cidx0, cidx1)
    ar = (ar0, ar1)
    br = (br0, br1)
    ov = (ov0, ov1)
    sa = (sa0, sa1)
    sb = (sb0, sb1)
    so = (so0, so1)

    wid = lax.axis_index("s") * NC + lax.axis_index("c")
    base = pl.multiple_of(wid * PER_W, 8)

    pltpu.sync_copy(w2_hbm, w2v)
    pltpu.sync_copy(b2_hbm, b2v)
    iota = lax.iota(jnp.int32, L)

    def offset(c):
        return pl.multiple_of(base + jnp.minimum(c * K, PER_W - K), 8)

    def fetch(c, s):
        st = offset(c)
        pltpu.sync_copy(row_hbm.at[pl.ds(st, K)], ridx[s])
        pltpu.sync_copy(col_hbm.at[pl.ds(st, K)], cidx[s])
        pltpu.async_copy(a_hbm.at[ridx[s]], ar[s], sa[s])
        pltpu.async_copy(b_hbm.at[cidx[s]], br[s], sb[s])

    def compute(c, s):
        arows, brows, outv = ar[s], br[s], ov[s]

        def edge(e, _):
            acc = b2v[...]
            for ci in range(HID // L):
                a = arows[e, pl.ds(ci * L, L)]
                b = brows[e, pl.ds(ci * L, L)]
                h = jnp.maximum(a + b, 0.0)
                acc = acc + h * w2v[ci]
            hsum[pl.ds(e * L, L)] = acc
            return 0

        lax.fori_loop(0, K, edge, 0)

        for g in range(K // L):
            flat = (g * L + iota) * L
            z = plsc.load_gather(hsum, [flat])
            for j in range(1, L):
                z = z + plsc.load_gather(hsum, [flat + j])
            outv[pl.ds(g * L, L)] = 1.0 / (1.0 + jnp.exp(-z))

        pltpu.async_copy(outv, out_hbm.at[pl.ds(offset(c), K)], so[s])

    fetch(0, 0)

    def pair(i, _):
        c0 = 2 * i
        for s in range(2):
            c = c0 + s

            @pl.when(c + 1 < N_CH)
            def _():
                fetch(c + 1, 1 - s)

            pltpu.make_async_copy(a_hbm.at[ridx[s]], ar[s], sa[s]).wait()
            pltpu.make_async_copy(b_hbm.at[cidx[s]], br[s], sb[s]).wait()

            @pl.when(c0 > 0)
            def _():
                pltpu.make_async_copy(ov[s], out_hbm.at[pl.ds(0, K)], so[s]).wait()

            compute(c, s)
        return 0

    lax.fori_loop(0, N_CH // 2, pair, 0)

    for s in range(2):
        pltpu.make_async_copy(ov[s], out_hbm.at[pl.ds(0, K)], so[s]).wait()


_edge_kernel = functools.partial(
    pl.kernel,
    out_type=jax.ShapeDtypeStruct((N_EDGES,), jnp.float32),
    mesh=plsc.VectorSubcoreMesh(core_axis_name="c", subcore_axis_name="s"),
    scratch_types=(
        [pltpu.VMEM((K,), jnp.int32)] * 4
        + [pltpu.VMEM((K, HID), jnp.float32)] * 4
        + [pltpu.VMEM((K,), jnp.float32)] * 2
        + [
            pltpu.VMEM((K * L,), jnp.float32),
            pltpu.VMEM((HID // L, L), jnp.float32),
            pltpu.VMEM((L,), jnp.float32),
        ]
        + [pltpu.SemaphoreType.DMA] * 6
    ),
    compiler_params=pltpu.CompilerParams(needs_layout_passes=False),
)(_edge_body)


def kernel(x, edge_index, W1, b1, W2, b2):
    row = edge_index[0].astype(jnp.int32)
    col = edge_index[1].astype(jnp.int32)
    w1a = W1[:, :IN_DIM]
    w1b = W1[:, IN_DIM:]
    a_tab, b_tab = _make_tables(x, w1a, w1b, b1.reshape(1, HID))
    w2r = W2.reshape(HID // L, L).astype(jnp.float32)
    b2v = jnp.broadcast_to(b2 / jnp.float32(L), (L,)).astype(jnp.float32)
    return _edge_kernel(a_tab, b_tab, row, col, w2r, b2v)


# speedup vs baseline: 5.2819x; 1.1458x over previous
import functools

import jax
import jax.numpy as jnp
from jax import lax
from jax.experimental import pallas as pl
from jax.experimental.pallas import tpu as pltpu
from jax.experimental.pallas import tpu_sc as plsc

IN_DIM = 128
HID = 128
N_NODES = 10000
N_EDGES = 320000

NC = 2
NS = 16
L = 16
NW = NC * NS
PER_W = N_EDGES // NW
K = 128
N_CHUNKS = -(-PER_W // K)


def _pack_rows(v):
    u = lax.bitcast_convert_type(v.astype(jnp.bfloat16), jnp.uint16)
    lo = u[:, :HID // 2].astype(jnp.uint32)
    hi = u[:, HID // 2:].astype(jnp.uint32)
    return lax.bitcast_convert_type(lo | (hi << 16), jnp.int32)


def _tables_body(x_ref, w1a_ref, w1b_ref, b1_ref, a_ref, b_ref):
    x = x_ref[...]
    dn = (((1,), (1,)), ((), ()))
    a = lax.dot_general(x, w1a_ref[...], dn, preferred_element_type=jnp.float32)
    b = lax.dot_general(x, w1b_ref[...], dn, preferred_element_type=jnp.float32)
    a_ref[...] = _pack_rows(a + b1_ref[...])
    b_ref[...] = _pack_rows(b)


def _make_tables(x, w1a, w1b, b1):
    blk = 1000
    grid = (N_NODES // blk,)
    return pl.pallas_call(
        _tables_body,
        grid=grid,
        in_specs=[
            pl.BlockSpec((blk, IN_DIM), lambda i: (i, 0)),
            pl.BlockSpec((HID, IN_DIM), lambda i: (0, 0)),
            pl.BlockSpec((HID, IN_DIM), lambda i: (0, 0)),
            pl.BlockSpec((1, HID), lambda i: (0, 0)),
        ],
        out_specs=[
            pl.BlockSpec((blk, HID // 2), lambda i: (i, 0)),
            pl.BlockSpec((blk, HID // 2), lambda i: (i, 0)),
        ],
        out_shape=[
            jax.ShapeDtypeStruct((N_NODES, HID // 2), jnp.int32),
            jax.ShapeDtypeStruct((N_NODES, HID // 2), jnp.int32),
        ],
    )(x, w1a, w1b, b1)


N_CH = 2 * (-(-PER_W // (2 * K)))


def _edge_body(a_hbm, b_hbm, row_hbm, col_hbm, w2_hbm, b2_hbm, out_hbm,
               ridx0, ridx1, cidx0, cidx1, ar0, ar1, br0, br1,
               ov0, ov1, hsum, w2v, b2v,
               sa0, sa1, sb0, sb1, so0, so1):
    ridx = (ridx0, ridx1)
    cidx = (cidx0, cidx1)
    ar = (ar0, ar1)
    br = (br0, br1)
    ov = (ov0, ov1)
    sa = (sa0, sa1)
    sb = (sb0, sb1)
    so = (so0, so1)

    wid = lax.axis_index("s") * NC + lax.axis_index("c")
    base = pl.multiple_of(wid * PER_W, 8)

    pltpu.sync_copy(w2_hbm, w2v)
    pltpu.sync_copy(b2_hbm, b2v)
    iota = lax.iota(jnp.int32, L)

    def offset(c):
        return pl.multiple_of(base + jnp.minimum(c * K, PER_W - K), 8)

    def fetch(c, s):
        st = offset(c)
        pltpu.sync_copy(row_hbm.at[pl.ds(st, K)], ridx[s])
        pltpu.sync_copy(col_hbm.at[pl.ds(st, K)], cidx[s])
        pltpu.async_copy(a_hbm.at[ridx[s]], ar[s], sa[s])
        pltpu.async_copy(b_hbm.at[cidx[s]], br[s], sb[s])

    def compute(c, s):
        arows, brows, outv = ar[s], br[s], ov[s]

        def edge(e, _):
            acc = b2v[...]
            for ci in range(HID // (2 * L)):
                aw = arows[e, pl.ds(ci * L, L)]
                bw = brows[e, pl.ds(ci * L, L)]
                a2 = plsc.bitcast(aw, jnp.bfloat16)
                b2_ = plsc.bitcast(bw, jnp.bfloat16)
                h = jnp.maximum(a2 + b2_, jnp.bfloat16(0))
                he, ho = plsc.unpack(h, format=plsc.PackFormat.INTERLEAVED)
                acc = acc + (he * w2v[2 * ci] + ho * w2v[2 * ci + 1])
            hsum[pl.ds(e * L, L)] = acc
            return 0

        lax.fori_loop(0, K, edge, 0)

        for g in range(K // L):
            flat = (g * L + iota) * L
            zs = [plsc.load_gather(hsum, [flat + j]) for j in range(L)]
            while len(zs) > 1:
                zs = [zs[i] + zs[i + 1] for i in range(0, len(zs), 2)]
            outv[pl.ds(g * L, L)] = 1.0 / (1.0 + jnp.exp(-zs[0]))

        pltpu.async_copy(outv, out_hbm.at[pl.ds(offset(c), K)], so[s])

    fetch(0, 0)

    def pair(i, _):
        c0 = 2 * i
        for s in range(2):
            c = c0 + s

            @pl.when(c + 1 < N_CH)
            def _():
                fetch(c + 1, 1 - s)

            pltpu.make_async_copy(a_hbm.at[ridx[s]], ar[s], sa[s]).wait()
            pltpu.make_async_copy(b_hbm.at[cidx[s]], br[s], sb[s]).wait()

            @pl.when(c0 > 0)
            def _():
                pltpu.make_async_copy(ov[s], out_hbm.at[pl.ds(0, K)], so[s]).wait()

            compute(c, s)
        return 0

    lax.fori_loop(0, N_CH // 2, pair, 0)

    for s in range(2):
        pltpu.make_async_copy(ov[s], out_hbm.at[pl.ds(0, K)], so[s]).wait()


_edge_kernel = functools.partial(
    pl.kernel,
    out_type=jax.ShapeDtypeStruct((N_EDGES,), jnp.float32),
    mesh=plsc.VectorSubcoreMesh(core_axis_name="c", subcore_axis_name="s"),
    scratch_types=(
        [pltpu.VMEM((K,), jnp.int32)] * 4
        + [pltpu.VMEM((K, HID // 2), jnp.int32)] * 4
        + [pltpu.VMEM((K,), jnp.float32)] * 2
        + [
            pltpu.VMEM((K * L,), jnp.float32),
            pltpu.VMEM((HID // L, L), jnp.float32),
            pltpu.VMEM((L,), jnp.float32),
        ]
        + [pltpu.SemaphoreType.DMA] * 6
    ),
    compiler_params=pltpu.CompilerParams(
        needs_layout_passes=False, use_tc_tiling_on_sc=False),
)(_edge_body)


def kernel(x, edge_index, W1, b1, W2, b2):
    row = edge_index[0].astype(jnp.int32)
    col = edge_index[1].astype(jnp.int32)
    w1a = W1[:, :IN_DIM]
    w1b = W1[:, IN_DIM:]
    a_tab, b_tab = _make_tables(x, w1a, w1b, b1.reshape(1, HID))
    w2r = (W2.reshape(2, HID // (2 * L), L).transpose(1, 0, 2)
           .reshape(HID // L, L).astype(jnp.float32))
    b2v = jnp.broadcast_to(b2 / jnp.float32(L), (L,)).astype(jnp.float32)
    return _edge_kernel(a_tab, b_tab, row, col, w2r, b2v)


# speedup vs baseline: 7.7640x; 1.4699x over previous
import functools

import jax
import jax.numpy as jnp
from jax import lax
from jax.experimental import pallas as pl
from jax.experimental.pallas import tpu as pltpu
from jax.experimental.pallas import tpu_sc as plsc

IN_DIM = 128
HID = 128
N_NODES = 10000
N_EDGES = 320000

NC = 2
NS = 16
L = 16
NW = NC * NS
PER_W = N_EDGES // NW
K = 128
N_CHUNKS = -(-PER_W // K)


def _pack_rows(v):
    u = lax.bitcast_convert_type(v.astype(jnp.bfloat16), jnp.uint16)
    lo = u[:, :HID // 2].astype(jnp.uint32)
    hi = u[:, HID // 2:].astype(jnp.uint32)
    return lax.bitcast_convert_type(lo | (hi << 16), jnp.int32)


def _tables_body(x_ref, w1a_ref, w1b_ref, b1_ref, a_ref, b_ref):
    x = x_ref[...]
    dn = (((1,), (1,)), ((), ()))
    a = lax.dot_general(x, w1a_ref[...], dn, preferred_element_type=jnp.float32)
    b = lax.dot_general(x, w1b_ref[...], dn, preferred_element_type=jnp.float32)
    a_ref[...] = _pack_rows(a + b1_ref[...])
    b_ref[...] = _pack_rows(b)


def _make_tables(x, w1a, w1b, b1):
    blk = 1000
    grid = (N_NODES // blk,)
    return pl.pallas_call(
        _tables_body,
        grid=grid,
        in_specs=[
            pl.BlockSpec((blk, IN_DIM), lambda i: (i, 0)),
            pl.BlockSpec((HID, IN_DIM), lambda i: (0, 0)),
            pl.BlockSpec((HID, IN_DIM), lambda i: (0, 0)),
            pl.BlockSpec((1, HID), lambda i: (0, 0)),
        ],
        out_specs=[
            pl.BlockSpec((blk, HID // 2), lambda i: (i, 0)),
            pl.BlockSpec((blk, HID // 2), lambda i: (i, 0)),
        ],
        out_shape=[
            jax.ShapeDtypeStruct((N_NODES, HID // 2), jnp.int32),
            jax.ShapeDtypeStruct((N_NODES, HID // 2), jnp.int32),
        ],
    )(x, w1a, w1b, b1)


N_CH = 2 * (-(-PER_W // (2 * K)))


def _edge_body(a_hbm, b_hbm, row_hbm, col_hbm, w2_hbm, b2_hbm, out_hbm,
               ridx0, ridx1, cidx0, cidx1, ar0, ar1, br0, br1,
               ov0, ov1, hsum, w2v, b2v,
               sa0, sa1, sb0, sb1, so0, so1):
    ridx = (ridx0, ridx1)
    cidx = (cidx0, cidx1)
    ar = (ar0, ar1)
    br = (br0, br1)
    ov = (ov0, ov1)
    sa = (sa0, sa1)
    sb = (sb0, sb1)
    so = (so0, so1)

    wid = lax.axis_index("s") * NC + lax.axis_index("c")
    base = pl.multiple_of(wid * PER_W, 8)

    pltpu.sync_copy(w2_hbm, w2v)
    pltpu.sync_copy(b2_hbm, b2v)
    iota = lax.iota(jnp.int32, L)

    def offset(c):
        return pl.multiple_of(base + jnp.minimum(c * K, PER_W - K), 8)

    def fetch(c, s):
        st = offset(c)
        pltpu.sync_copy(row_hbm.at[pl.ds(st, K)], ridx[s])
        pltpu.sync_copy(col_hbm.at[pl.ds(st, K)], cidx[s])
        pltpu.async_copy(a_hbm.at[ridx[s]], ar[s], sa[s])
        pltpu.async_copy(b_hbm.at[cidx[s]], br[s], sb[s])

    def compute(c, s):
        arows, brows, outv = ar[s], br[s], ov[s]

        @plsc.parallel_loop(0, K, unroll=4)
        def edge(e):
            acc2 = jnp.zeros((2 * L,), jnp.bfloat16)
            for ci in range(HID // (2 * L)):
                aw = arows[e, pl.ds(ci * L, L)]
                bw = brows[e, pl.ds(ci * L, L)]
                a2 = plsc.bitcast(aw, jnp.bfloat16)
                b2_ = plsc.bitcast(bw, jnp.bfloat16)
                h = jnp.maximum(a2 + b2_, jnp.bfloat16(0))
                acc2 = acc2 + h * w2v[ci]
            he, ho = plsc.unpack(acc2, format=plsc.PackFormat.INTERLEAVED)
            hsum[pl.ds(e * L, L)] = he + ho + b2v[...]

        for g in range(K // L):
            flat = (g * L + iota) * L
            zs = [plsc.load_gather(hsum, [flat + j]) for j in range(L)]
            while len(zs) > 1:
                zs = [zs[i] + zs[i + 1] for i in range(0, len(zs), 2)]
            outv[pl.ds(g * L, L)] = 1.0 / (1.0 + jnp.exp(-zs[0]))

        pltpu.async_copy(outv, out_hbm.at[pl.ds(offset(c), K)], so[s])

    fetch(0, 0)

    def pair(i, _):
        c0 = 2 * i
        for s in range(2):
            c = c0 + s

            @pl.when(c + 1 < N_CH)
            def _():
                fetch(c + 1, 1 - s)

            pltpu.make_async_copy(a_hbm.at[ridx[s]], ar[s], sa[s]).wait()
            pltpu.make_async_copy(b_hbm.at[cidx[s]], br[s], sb[s]).wait()

            @pl.when(c0 > 0)
            def _():
                pltpu.make_async_copy(ov[s], out_hbm.at[pl.ds(0, K)], so[s]).wait()

            compute(c, s)
        return 0

    lax.fori_loop(0, N_CH // 2, pair, 0)

    for s in range(2):
        pltpu.make_async_copy(ov[s], out_hbm.at[pl.ds(0, K)], so[s]).wait()


_edge_kernel = functools.partial(
    pl.kernel,
    out_type=jax.ShapeDtypeStruct((N_EDGES,), jnp.float32),
    mesh=plsc.VectorSubcoreMesh(core_axis_name="c", subcore_axis_name="s"),
    scratch_types=(
        [pltpu.VMEM((K,), jnp.int32)] * 4
        + [pltpu.VMEM((K, HID // 2), jnp.int32)] * 4
        + [pltpu.VMEM((K,), jnp.float32)] * 2
        + [
            pltpu.VMEM((K * L,), jnp.float32),
            pltpu.VMEM((HID // (2 * L), 2 * L), jnp.bfloat16),
            pltpu.VMEM((L,), jnp.float32),
        ]
        + [pltpu.SemaphoreType.DMA] * 6
    ),
    compiler_params=pltpu.CompilerParams(
        needs_layout_passes=False, use_tc_tiling_on_sc=False),
)(_edge_body)


def kernel(x, edge_index, W1, b1, W2, b2):
    row = edge_index[0].astype(jnp.int32)
    col = edge_index[1].astype(jnp.int32)
    w1a = W1[:, :IN_DIM]
    w1b = W1[:, IN_DIM:]
    a_tab, b_tab = _make_tables(x, w1a, w1b, b1.reshape(1, HID))
    w2r = (W2.reshape(2, HID // (2 * L), L).transpose(1, 2, 0)
           .reshape(HID // (2 * L), 2 * L).astype(jnp.bfloat16))
    b2v = jnp.broadcast_to(b2 / jnp.float32(L), (L,)).astype(jnp.float32)
    return _edge_kernel(a_tab, b_tab, row, col, w2r, b2v)


# speedup vs baseline: 11.3238x; 1.4585x over previous
import functools

import jax
import jax.numpy as jnp
from jax import lax
from jax.experimental import pallas as pl
from jax.experimental.pallas import tpu as pltpu
from jax.experimental.pallas import tpu_sc as plsc

IN_DIM = 128
HID = 128
N_NODES = 10000
N_EDGES = 320000

NC = 2
NS = 16
L = 16
NW = NC * NS
PER_W = N_EDGES // NW
K = 128
N_CHUNKS = -(-PER_W // K)


def _pack_rows(v):
    u = lax.bitcast_convert_type(v.astype(jnp.bfloat16), jnp.uint16)
    lo = u[:, :HID // 2].astype(jnp.uint32)
    hi = u[:, HID // 2:].astype(jnp.uint32)
    return lax.bitcast_convert_type(lo | (hi << 16), jnp.int32)


def _tables_body(x_ref, w1a_ref, w1b_ref, b1_ref, a_ref, b_ref):
    x = x_ref[...]
    dn = (((1,), (1,)), ((), ()))
    a = lax.dot_general(x, w1a_ref[...], dn, preferred_element_type=jnp.float32)
    b = lax.dot_general(x, w1b_ref[...], dn, preferred_element_type=jnp.float32)
    a_ref[...] = _pack_rows(a + b1_ref[...])
    b_ref[...] = _pack_rows(b)


def _make_tables(x, w1a, w1b, b1):
    blk = 1000
    grid = (N_NODES // blk,)
    return pl.pallas_call(
        _tables_body,
        grid=grid,
        in_specs=[
            pl.BlockSpec((blk, IN_DIM), lambda i: (i, 0)),
            pl.BlockSpec((HID, IN_DIM), lambda i: (0, 0)),
            pl.BlockSpec((HID, IN_DIM), lambda i: (0, 0)),
            pl.BlockSpec((1, HID), lambda i: (0, 0)),
        ],
        out_specs=[
            pl.BlockSpec((blk, HID // 2), lambda i: (i, 0)),
            pl.BlockSpec((blk, HID // 2), lambda i: (i, 0)),
        ],
        out_shape=[
            jax.ShapeDtypeStruct((N_NODES, HID // 2), jnp.int32),
            jax.ShapeDtypeStruct((N_NODES, HID // 2), jnp.int32),
        ],
    )(x, w1a, w1b, b1)


N_CH = 2 * (-(-PER_W // (2 * K)))


def _edge_body(a_hbm, b_hbm, row_hbm, col_hbm, w2_hbm, b2_hbm, out_hbm,
               ridx_all, cidx_all, ar0, ar1, br0, br1,
               ov0, ov1, hsum, w2v, b2v,
               sa0, sa1, sb0, sb1, so0, so1):
    ar = (ar0, ar1)
    br = (br0, br1)
    ov = (ov0, ov1)
    sa = (sa0, sa1)
    sb = (sb0, sb1)
    so = (so0, so1)

    wid = lax.axis_index("s") * NC + lax.axis_index("c")
    base = pl.multiple_of(wid * PER_W, 8)

    pltpu.sync_copy(w2_hbm, w2v)
    pltpu.sync_copy(b2_hbm, b2v)
    pltpu.sync_copy(row_hbm.at[pl.ds(base, PER_W)], ridx_all)
    pltpu.sync_copy(col_hbm.at[pl.ds(base, PER_W)], cidx_all)
    iota = lax.iota(jnp.int32, L)

    def local_off(c):
        return pl.multiple_of(jnp.minimum(c * K, PER_W - K), 8)

    def offset(c):
        return pl.multiple_of(base + jnp.minimum(c * K, PER_W - K), 8)

    def fetch(c, s):
        sl = local_off(c)
        pltpu.async_copy(a_hbm.at[ridx_all.at[pl.ds(sl, K)]], ar[s], sa[s])
        pltpu.async_copy(b_hbm.at[cidx_all.at[pl.ds(sl, K)]], br[s], sb[s])

    def compute(c, s):
        arows, brows, outv = ar[s], br[s], ov[s]

        @plsc.parallel_loop(0, K, unroll=4)
        def edge(e):
            acc2 = jnp.zeros((2 * L,), jnp.bfloat16)
            for ci in range(HID // (2 * L)):
                aw = arows[e, pl.ds(ci * L, L)]
                bw = brows[e, pl.ds(ci * L, L)]
                a2 = plsc.bitcast(aw, jnp.bfloat16)
                b2_ = plsc.bitcast(bw, jnp.bfloat16)
                h = jnp.maximum(a2 + b2_, jnp.bfloat16(0))
                acc2 = acc2 + h * w2v[ci]
            he, ho = plsc.unpack(acc2, format=plsc.PackFormat.INTERLEAVED)
            hsum[pl.ds(e * L, L)] = he + ho + b2v[...]

        for g in range(K // L):
            flat = (g * L + iota) * L
            zs = [plsc.load_gather(hsum, [flat + j]) for j in range(L)]
            while len(zs) > 1:
                zs = [zs[i] + zs[i + 1] for i in range(0, len(zs), 2)]
            outv[pl.ds(g * L, L)] = 1.0 / (1.0 + jnp.exp(-zs[0]))

        pltpu.async_copy(outv, out_hbm.at[pl.ds(offset(c), K)], so[s])

    fetch(0, 0)

    def pair(i, _):
        c0 = 2 * i
        for s in range(2):
            c = c0 + s

            @pl.when(c + 1 < N_CH)
            def _():
                fetch(c + 1, 1 - s)

            pltpu.make_async_copy(
                a_hbm.at[ridx_all.at[pl.ds(0, K)]], ar[s], sa[s]).wait()
            pltpu.make_async_copy(
                b_hbm.at[cidx_all.at[pl.ds(0, K)]], br[s], sb[s]).wait()

            @pl.when(c0 > 0)
            def _():
                pltpu.make_async_copy(ov[s], out_hbm.at[pl.ds(0, K)], so[s]).wait()

            compute(c, s)
        return 0

    lax.fori_loop(0, N_CH // 2, pair, 0)

    for s in range(2):
        pltpu.make_async_copy(ov[s], out_hbm.at[pl.ds(0, K)], so[s]).wait()


_edge_kernel = functools.partial(
    pl.kernel,
    out_type=jax.ShapeDtypeStruct((N_EDGES,), jnp.float32),
    mesh=plsc.VectorSubcoreMesh(core_axis_name="c", subcore_axis_name="s"),
    scratch_types=(
        [pltpu.VMEM((PER_W,), jnp.int32)] * 2
        + [pltpu.VMEM((K, HID // 2), jnp.int32)] * 4
        + [pltpu.VMEM((K,), jnp.float32)] * 2
        + [
            pltpu.VMEM((K * L,), jnp.float32),
            pltpu.VMEM((HID // (2 * L), 2 * L), jnp.bfloat16),
            pltpu.VMEM((L,), jnp.float32),
        ]
        + [pltpu.SemaphoreType.DMA] * 6
    ),
    compiler_params=pltpu.CompilerParams(
        needs_layout_passes=False, use_tc_tiling_on_sc=False),
)(_edge_body)


def kernel(x, edge_index, W1, b1, W2, b2):
    row = edge_index[0].astype(jnp.int32)
    col = edge_index[1].astype(jnp.int32)
    w1a = W1[:, :IN_DIM]
    w1b = W1[:, IN_DIM:]
    a_tab, b_tab = _make_tables(x, w1a, w1b, b1.reshape(1, HID))
    w2r = (W2.reshape(2, HID // (2 * L), L).transpose(1, 2, 0)
           .reshape(HID // (2 * L), 2 * L).astype(jnp.bfloat16))
    b2v = jnp.broadcast_to(b2 / jnp.float32(L), (L,)).astype(jnp.float32)
    return _edge_kernel(a_tab, b_tab, row, col, w2r, b2v)
